# baseline (speedup 1.0000x reference)
import functools
import jax
import jax.numpy as jnp
from jax import lax
from jax.experimental import pallas as pl
from jax.experimental.pallas import tpu as pltpu
from jax.experimental.pallas import tpu_sc as plsc

NCORES = 2
NSUB = 16
NW = NCORES * NSUB
CH = 128


def _sc_mesh():
    return plsc.VectorSubcoreMesh(
        core_axis_name="c", subcore_axis_name="s",
        num_cores=NCORES, num_subcores=NSUB)


def _make_weights_kernel(cpt, cnt_pad, pad_seg):
    zpt = cnt_pad // NSUB

    @functools.partial(
        pl.kernel,
        out_type=jax.ShapeDtypeStruct((NW, cpt, CH), jnp.float32),
        mesh=_sc_mesh(),
        scratch_types=[
            pltpu.VMEM((NCORES, cpt, CH), jnp.int32),
            pltpu.VMEM((CH,), jnp.float32),
            pltpu.VMEM((CH,), jnp.float32),
            pltpu.VMEM((CH,), jnp.float32),
            pltpu.VMEM((zpt,), jnp.float32),
            pltpu.VMEM_SHARED((cnt_pad,), jnp.float32),
        ],
    )
    def k(seg_hbm, w_hbm, seg_v, ones_v, cvals_v, w_v, zz_v, cnt_sh):
        cid = lax.axis_index("c")
        sid = lax.axis_index("s")
        for l in range(CH // 16):
            ones_v[pl.ds(l * 16, 16)] = jnp.full((16,), 1.0, jnp.float32)

        def zfill(i, _):
            zz_v[pl.ds(i * 16, 16)] = jnp.zeros((16,), jnp.float32)
            return 0
        lax.fori_loop(0, zpt // 16, zfill, 0)
        for t in range(NCORES):
            pltpu.sync_copy(seg_hbm.at[NCORES * sid + t], seg_v.at[t])
        pltpu.sync_copy(zz_v, cnt_sh.at[pl.ds(sid * zpt, zpt)])
        plsc.subcore_barrier()

        def count_body(j, _):
            for t in range(NCORES):
                pltpu.sync_copy(ones_v, cnt_sh.at[seg_v.at[t, j]], add=True)
            return 0
        lax.fori_loop(0, cpt, count_body, 0)
        plsc.subcore_barrier()

        def w_body(j, _):
            pltpu.sync_copy(cnt_sh.at[seg_v.at[cid, j]], cvals_v)
            for l in range(CH // 16):
                sl = pl.ds(l * 16, 16)
                c = cvals_v[sl]
                s = seg_v[cid, j, sl]
                w = 1.0 / jnp.maximum(c, 1.0)
                w_v[sl] = jnp.where(s >= pad_seg, 0.0, w)
            pltpu.sync_copy(w_v, w_hbm.at[NCORES * sid + cid, j])
            return 0
        lax.fori_loop(0, cpt, w_body, 0)

    return k


def _make_agg_kernel(n_pad, d, cpt):
    zs = n_pad // NSUB

    @functools.partial(
        pl.kernel,
        out_type=jax.ShapeDtypeStruct((NCORES, n_pad, d), jnp.float32),
        mesh=_sc_mesh(),
        scratch_types=[
            pltpu.VMEM((cpt, CH), jnp.int32),
            pltpu.VMEM((2, CH), jnp.int32),
            pltpu.VMEM((2, CH), jnp.float32),
            pltpu.VMEM((2, CH, d // 2), jnp.int32),
            pltpu.VMEM((CH, d), jnp.float32),
            pltpu.SemaphoreType.DMA((2,)),
            pltpu.SemaphoreType.DMA((2,)),
            pltpu.SemaphoreType.DMA((2,)),
            pltpu.VMEM_SHARED((n_pad, d), jnp.float32),
        ],
        compiler_params=pltpu.CompilerParams(needs_layout_passes=False,
                                             use_tc_tiling_on_sc=False),
    )
    def k(xw_hbm, gidx_hbm, dst_hbm, w_hbm, out_hbm,
          gidx_v, dst_r, w_r, rows_b, rowsf, sem_g, sem_d, sem_w, acc_sh):
        cid = lax.axis_index("c")
        sid = lax.axis_index("s")
        wid = NCORES * sid + cid
        pltpu.sync_copy(gidx_hbm.at[wid], gidx_v)

        def zfill(i, _):
            for g in range(d // 16):
                rowsf[i, pl.ds(g * 16, 16)] = jnp.zeros((16,), jnp.float32)
            return 0
        lax.fori_loop(0, CH, zfill, 0)
        nfull, rem = divmod(zs, CH)
        for q in range(nfull):
            pltpu.sync_copy(rowsf,
                            acc_sh.at[pl.ds(sid * zs + q * CH, CH)])
        if rem:
            pltpu.sync_copy(rowsf.at[pl.ds(0, rem)],
                            acc_sh.at[pl.ds(sid * zs + nfull * CH, rem)])
        plsc.subcore_barrier()

        def prefetch(j, p):
            pltpu.async_copy(xw_hbm.at[gidx_v.at[j]], rows_b.at[p],
                             sem_g.at[p])
            pltpu.async_copy(dst_hbm.at[wid, j], dst_r.at[p], sem_d.at[p])
            pltpu.async_copy(w_hbm.at[wid, j], w_r.at[p], sem_w.at[p])

        def prime(p0, _):
            prefetch(p0, p0)
            return 0
        lax.fori_loop(0, 2, prime, 0)

        def chunk_body(j, _):
            p = lax.rem(j, 2)
            pltpu.make_async_copy(xw_hbm.at[gidx_v.at[j]], rows_b.at[p],
                                  sem_g.at[p]).wait()
            pltpu.make_async_copy(w_hbm.at[wid, j], w_r.at[p],
                                  sem_w.at[p]).wait()
            pv = jnp.full((16,), p, jnp.int32)
            hmask = jnp.full((16,), -65536, jnp.int32)
            lane2 = 2 * lax.iota(jnp.int32, 16)

            def row_body(i4, _):
                for u in range(4):
                    i = i4 * 4 + u
                    iv = jnp.full((16,), i, jnp.int32)
                    wspl = plsc.load_gather(w_r, [pv, iv])
                    for g in range(d // 32):
                        v = rows_b[p, i, pl.ds(g * 16, 16)]
                        lo = plsc.bitcast(jnp.left_shift(v, 16), jnp.float32)
                        hi = plsc.bitcast(jnp.bitwise_and(v, hmask),
                                          jnp.float32)
                        plsc.store_scatter(rowsf, [iv, g * 32 + lane2],
                                           lo * wspl)
                        plsc.store_scatter(rowsf, [iv, g * 32 + 1 + lane2],
                                           hi * wspl)
                return 0
            lax.fori_loop(0, CH // 4, row_body, 0)

            pltpu.make_async_copy(dst_hbm.at[wid, j], dst_r.at[p],
                                  sem_d.at[p]).wait()
            pltpu.sync_copy(rowsf, acc_sh.at[dst_r.at[p]], add=True)

            @pl.when(j + 2 < cpt)
            def _():
                prefetch(j + 2, p)
            return 0
        lax.fori_loop(0, cpt, chunk_body, 0)
        plsc.subcore_barrier()
        pltpu.sync_copy(acc_sh.at[pl.ds(sid * zs, zs)],
                        out_hbm.at[cid, pl.ds(sid * zs, zs)])

    return k


def _relmm(x, W, bn):
    n, din = x.shape
    r, _, dout = W.shape
    h = dout // 2

    def body(x_ref, w_ref, out_ref):
        xb = x_ref[...]
        for ri in range(r):
            y = jnp.dot(xb, w_ref[ri], preferred_element_type=jnp.float32)
            out_ref[:, ri, :] = y.astype(jnp.bfloat16)

    return pl.pallas_call(
        body,
        grid=(n // bn,),
        in_specs=[
            pl.BlockSpec((bn, din), lambda i: (i, 0)),
            pl.BlockSpec((r, din, dout), lambda i: (0, 0, 0)),
        ],
        out_specs=pl.BlockSpec((bn, r, dout), lambda i: (i, 0, 0)),
        out_shape=jax.ShapeDtypeStruct((n, r, dout), jnp.bfloat16),
    )(x, W)


def _post(p0, p1, x, root, b, bn, relu):
    n, din = x.shape
    dout = root.shape[1]

    def body(p0_ref, p1_ref, x_ref, root_ref, b_ref, out_ref):
        h = (p0_ref[...] + p1_ref[...]
             + jnp.dot(x_ref[...], root_ref[...],
                       preferred_element_type=jnp.float32)
             + b_ref[...])
        if relu:
            h = jnp.maximum(h, 0.0)
        out_ref[...] = h

    return pl.pallas_call(
        body,
        grid=(n // bn,),
        in_specs=[
            pl.BlockSpec((bn, dout), lambda i: (i, 0)),
            pl.BlockSpec((bn, dout), lambda i: (i, 0)),
            pl.BlockSpec((bn, din), lambda i: (i, 0)),
            pl.BlockSpec((din, dout), lambda i: (0, 0)),
            pl.BlockSpec((1, dout), lambda i: (0, 0)),
        ],
        out_specs=pl.BlockSpec((bn, dout), lambda i: (i, 0)),
        out_shape=jax.ShapeDtypeStruct((n, dout), jnp.float32),
    )(p0, p1, x, root, b)


def _post_final(p0, p1, x, root, b, wc, bc, bn):
    n, din = x.shape
    dout = root.shape[1]
    dc = wc.shape[1]

    def body(p0_ref, p1_ref, x_ref, root_ref, b_ref, wc_ref, bc_ref,
             h_ref, lg_ref):
        h = (p0_ref[...] + p1_ref[...]
             + jnp.dot(x_ref[...], root_ref[...],
                       preferred_element_type=jnp.float32)
             + b_ref[...])
        h_ref[...] = h
        lg_ref[...] = jnp.dot(h, wc_ref[...],
                              preferred_element_type=jnp.float32) + bc_ref[...]

    return pl.pallas_call(
        body,
        grid=(n // bn,),
        in_specs=[
            pl.BlockSpec((bn, dout), lambda i: (i, 0)),
            pl.BlockSpec((bn, dout), lambda i: (i, 0)),
            pl.BlockSpec((bn, din), lambda i: (i, 0)),
            pl.BlockSpec((din, dout), lambda i: (0, 0)),
            pl.BlockSpec((1, dout), lambda i: (0, 0)),
            pl.BlockSpec((dout, dc), lambda i: (0, 0)),
            pl.BlockSpec((1, dc), lambda i: (0, 0)),
        ],
        out_specs=[
            pl.BlockSpec((bn, dout), lambda i: (i, 0)),
            pl.BlockSpec((bn, dc), lambda i: (i, 0)),
        ],
        out_shape=[
            jax.ShapeDtypeStruct((n, dout), jnp.float32),
            jax.ShapeDtypeStruct((n, dc), jnp.float32),
        ],
    )(p0, p1, x, root, b, wc, bc)


def kernel(x, edge_index, edge_type, W1, root1, b1, W2, root2, b2, Wc, bc):
    n, din = x.shape
    r = W1.shape[0]
    d = W1.shape[2]
    e = edge_type.shape[0]
    nc = Wc.shape[1]

    cpt = -(-e // (NW * CH))
    e_pad = NW * cpt * CH
    pad = e_pad - e
    pad_seg = n * r
    cnt_pad = pad_seg + 1
    if cnt_pad % (NSUB * 16):
        cnt_pad += NSUB * 16 - cnt_pad % (NSUB * 16)
    n_pad = NSUB * (-(-n // (NSUB * 8)) * 8)

    src = edge_index[0]
    dst = edge_index[1]
    et = edge_type.astype(jnp.int32)
    gidx = src * r + et
    seg = dst * r + et
    i0 = jnp.zeros((pad,), jnp.int32)
    gidx3 = jnp.concatenate([gidx, i0]).reshape(NW, cpt, CH)
    seg3 = jnp.concatenate([seg, jnp.full((pad,), pad_seg, jnp.int32)]
                           ).reshape(NW, cpt, CH)
    dst3 = jnp.concatenate([dst, i0]).reshape(NW, cpt, CH)

    w3 = _make_weights_kernel(cpt, cnt_pad, pad_seg)(seg3)
    agg = _make_agg_kernel(n_pad, d, cpt)

    bn = 1000
    b1_2d = b1.reshape(1, d)
    b2_2d = b2.reshape(1, d)
    wc_pad = jnp.zeros((d, d), jnp.float32).at[:, :nc].set(Wc)
    bc_pad = jnp.zeros((1, d), jnp.float32).at[:, :nc].set(bc)

    xw1 = jax.lax.bitcast_convert_type(
        _relmm(x, W1, bn).reshape(n * r, d // 2, 2), jnp.int32)
    p1 = agg(xw1, gidx3, dst3, w3)
    h = _post(p1[0, :n], p1[1, :n], x, root1, b1_2d, bn, relu=True)

    xw2 = jax.lax.bitcast_convert_type(
        _relmm(h, W2, bn).reshape(n * r, d // 2, 2), jnp.int32)
    p2 = agg(xw2, gidx3, dst3, w3)
    h2, logits_pad = _post_final(p2[0, :n], p2[1, :n], h, root2, b2_2d,
                                 wc_pad, bc_pad, bn)
    return (h2, logits_pad[:, :nc])


# speedup vs baseline: 3.5829x; 3.5829x over previous
import functools
import jax
import jax.numpy as jnp
from jax import lax
from jax.experimental import pallas as pl
from jax.experimental.pallas import tpu as pltpu
from jax.experimental.pallas import tpu_sc as plsc

NCORES = 2
NSUB = 16
NW = NCORES * NSUB
CH = 128


def _sc_mesh():
    return plsc.VectorSubcoreMesh(
        core_axis_name="c", subcore_axis_name="s",
        num_cores=NCORES, num_subcores=NSUB)


def _make_weights_kernel(cpt, cnt_pad, pad_seg):
    zpt = cnt_pad // NSUB

    @functools.partial(
        pl.kernel,
        out_type=jax.ShapeDtypeStruct((NW, cpt, CH), jnp.float32),
        mesh=_sc_mesh(),
        scratch_types=[
            pltpu.VMEM((NCORES, cpt, CH), jnp.int32),
            pltpu.VMEM((CH,), jnp.float32),
            pltpu.VMEM((2, CH), jnp.float32),
            pltpu.VMEM((2, CH), jnp.float32),
            pltpu.VMEM((zpt,), jnp.float32),
            pltpu.SemaphoreType.DMA,
            pltpu.SemaphoreType.DMA((2,)),
            pltpu.SemaphoreType.DMA((2,)),
            pltpu.VMEM_SHARED((cnt_pad,), jnp.float32),
        ],
    )
    def k(seg_hbm, w_hbm, seg_v, ones_v, cval_r, wout_r, zz_v,
          sem_c, sem_cv, sem_wo, cnt_sh):
        cid = lax.axis_index("c")
        sid = lax.axis_index("s")
        row = NCORES * sid + cid
        for l in range(CH // 16):
            ones_v[pl.ds(l * 16, 16)] = jnp.full((16,), 1.0, jnp.float32)

        def zfill(i, _):
            zz_v[pl.ds(i * 16, 16)] = jnp.zeros((16,), jnp.float32)
            return 0
        lax.fori_loop(0, zpt // 16, zfill, 0)
        for t in range(NCORES):
            pltpu.sync_copy(seg_hbm.at[NCORES * sid + t], seg_v.at[t])
        pltpu.sync_copy(zz_v, cnt_sh.at[pl.ds(sid * zpt, zpt)])
        plsc.subcore_barrier()

        def count_body(j, _):
            for t in range(NCORES):
                pltpu.async_copy(ones_v, cnt_sh.at[seg_v.at[t, j]], sem_c,
                                 add=True)
            return 0
        lax.fori_loop(0, cpt, count_body, 0)

        def count_drain(j, _):
            for t in range(NCORES):
                pltpu.make_async_copy(ones_v, cnt_sh.at[seg_v.at[t, j]],
                                      sem_c).wait()
            return 0
        lax.fori_loop(0, cpt, count_drain, 0)
        plsc.subcore_barrier()

        def wpre(j, p):
            pltpu.async_copy(cnt_sh.at[seg_v.at[cid, j]], cval_r.at[p],
                             sem_cv.at[p])

        def wprime(p0, _):
            wpre(p0, p0)
            return 0
        lax.fori_loop(0, 2, wprime, 0)

        def w_body(j, _):
            p = lax.rem(j, 2)
            pltpu.make_async_copy(cnt_sh.at[seg_v.at[cid, j]], cval_r.at[p],
                                  sem_cv.at[p]).wait()

            @pl.when(j >= 2)
            def _():
                pltpu.make_async_copy(wout_r.at[p], w_hbm.at[row, j],
                                      sem_wo.at[p]).wait()
            for l in range(CH // 16):
                sl = pl.ds(l * 16, 16)
                c = cval_r[p, sl]
                s = seg_v[cid, j, sl]
                w = 1.0 / jnp.maximum(c, 1.0)
                wout_r[p, sl] = jnp.where(s >= pad_seg, 0.0, w)
            pltpu.async_copy(wout_r.at[p], w_hbm.at[row, j], sem_wo.at[p])

            @pl.when(j + 2 < cpt)
            def _():
                wpre(j + 2, p)
            return 0
        lax.fori_loop(0, cpt, w_body, 0)

        def wdrain(q, _):
            j = cpt - 2 + q
            p = lax.rem(j, 2)
            pltpu.make_async_copy(wout_r.at[p], w_hbm.at[row, j],
                                  sem_wo.at[p]).wait()
            return 0
        lax.fori_loop(0, 2, wdrain, 0)

    return k


def _make_agg_kernel(n_pad, d, cpt):
    zs = n_pad // NSUB

    @functools.partial(
        pl.kernel,
        out_type=jax.ShapeDtypeStruct((NCORES, n_pad, d), jnp.float32),
        mesh=_sc_mesh(),
        scratch_types=[
            pltpu.VMEM((cpt, CH), jnp.int32),
            pltpu.VMEM((2, CH), jnp.int32),
            pltpu.VMEM((2, CH), jnp.float32),
            pltpu.VMEM((2, CH, d), jnp.float32),
            pltpu.SemaphoreType.DMA((2,)),
            pltpu.SemaphoreType.DMA((2,)),
            pltpu.SemaphoreType.DMA((2,)),
            pltpu.VMEM_SHARED((n_pad, d), jnp.float32),
        ],
        compiler_params=pltpu.CompilerParams(needs_layout_passes=False),
    )
    def k(xw_hbm, gidx_hbm, dst_hbm, w_hbm, out_hbm,
          gidx_v, dst_r, w_r, rows_b, sem_g, sem_d, sem_w, acc_sh):
        cid = lax.axis_index("c")
        sid = lax.axis_index("s")
        wid = NCORES * sid + cid
        pltpu.sync_copy(gidx_hbm.at[wid], gidx_v)

        def zfill(i, _):
            for g in range(d // 16):
                rows_b[0, i, pl.ds(g * 16, 16)] = jnp.zeros((16,), jnp.float32)
            return 0
        lax.fori_loop(0, CH, zfill, 0)
        nfull, rem = divmod(zs, CH)
        for q in range(nfull):
            pltpu.sync_copy(rows_b.at[0],
                            acc_sh.at[pl.ds(sid * zs + q * CH, CH)])
        if rem:
            pltpu.sync_copy(rows_b.at[0, pl.ds(0, rem)],
                            acc_sh.at[pl.ds(sid * zs + nfull * CH, rem)])
        plsc.subcore_barrier()

        def prefetch(j, p):
            pltpu.async_copy(xw_hbm.at[gidx_v.at[j]], rows_b.at[p],
                             sem_g.at[p])
            pltpu.async_copy(dst_hbm.at[wid, j], dst_r.at[p], sem_d.at[p])
            pltpu.async_copy(w_hbm.at[wid, j], w_r.at[p], sem_w.at[p])

        def prime(p0, _):
            prefetch(p0, p0)
            return 0
        lax.fori_loop(0, 2, prime, 0)

        def chunk_body(j, _):
            p = lax.rem(j, 2)
            pltpu.make_async_copy(xw_hbm.at[gidx_v.at[j]], rows_b.at[p],
                                  sem_g.at[p]).wait()
            pltpu.make_async_copy(w_hbm.at[wid, j], w_r.at[p],
                                  sem_w.at[p]).wait()
            pv = jnp.full((16,), p, jnp.int32)

            def row_body(i4, _):
                for u in range(4):
                    i = i4 * 4 + u
                    wspl = plsc.load_gather(
                        w_r, [pv, jnp.full((16,), i, jnp.int32)])
                    for g in range(d // 16):
                        sl = pl.ds(g * 16, 16)
                        rows_b[p, i, sl] = rows_b[p, i, sl] * wspl
                return 0
            lax.fori_loop(0, CH // 4, row_body, 0)

            pltpu.make_async_copy(dst_hbm.at[wid, j], dst_r.at[p],
                                  sem_d.at[p]).wait()
            pltpu.sync_copy(rows_b.at[p], acc_sh.at[dst_r.at[p]], add=True)

            @pl.when(j + 2 < cpt)
            def _():
                prefetch(j + 2, p)
            return 0
        lax.fori_loop(0, cpt, chunk_body, 0)
        plsc.subcore_barrier()
        pltpu.sync_copy(acc_sh.at[pl.ds(sid * zs, zs)],
                        out_hbm.at[cid, pl.ds(sid * zs, zs)])

    return k


def _relmm(x, W, bn):
    n, din = x.shape
    r, _, dout = W.shape
    h = dout // 2

    def body(x_ref, w_ref, out_ref):
        xb = x_ref[...]
        for ri in range(r):
            out_ref[:, ri, :] = jnp.dot(xb, w_ref[ri],
                                        preferred_element_type=jnp.float32)

    return pl.pallas_call(
        body,
        grid=(n // bn,),
        in_specs=[
            pl.BlockSpec((bn, din), lambda i: (i, 0)),
            pl.BlockSpec((r, din, dout), lambda i: (0, 0, 0)),
        ],
        out_specs=pl.BlockSpec((bn, r, dout), lambda i: (i, 0, 0)),
        out_shape=jax.ShapeDtypeStruct((n, r, dout), jnp.float32),
    )(x, W)


def _post_relmm(p, x, root, b, W2, bn):
    n, din = x.shape
    n_pad = p.shape[1]
    dout = root.shape[1]
    r = W2.shape[0]

    def body(p_ref, x_ref, root_ref, b_ref, w2_ref, h_ref, xw_ref):
        h = (p_ref[0] + p_ref[1]
             + jnp.dot(x_ref[...], root_ref[...],
                       preferred_element_type=jnp.float32)
             + b_ref[...])
        h = jnp.maximum(h, 0.0)
        h_ref[...] = h
        for ri in range(r):
            xw_ref[:, ri, :] = jnp.dot(h, w2_ref[ri],
                                       preferred_element_type=jnp.float32)

    return pl.pallas_call(
        body,
        grid=(n // bn,),
        in_specs=[
            pl.BlockSpec((2, bn, dout), lambda i: (0, i, 0)),
            pl.BlockSpec((bn, din), lambda i: (i, 0)),
            pl.BlockSpec((din, dout), lambda i: (0, 0)),
            pl.BlockSpec((1, dout), lambda i: (0, 0)),
            pl.BlockSpec((r, dout, dout), lambda i: (0, 0, 0)),
        ],
        out_specs=[
            pl.BlockSpec((bn, dout), lambda i: (i, 0)),
            pl.BlockSpec((bn, r, dout), lambda i: (i, 0, 0)),
        ],
        out_shape=[
            jax.ShapeDtypeStruct((n, dout), jnp.float32),
            jax.ShapeDtypeStruct((n, r, dout), jnp.float32),
        ],
    )(p, x, root, b, W2)


def _post_final(p, x, root, b, wc, bc, bn):
    n, din = x.shape
    dout = root.shape[1]
    dc = wc.shape[1]

    def body(p_ref, x_ref, root_ref, b_ref, wc_ref, bc_ref,
             h_ref, lg_ref):
        h = (p_ref[0] + p_ref[1]
             + jnp.dot(x_ref[...], root_ref[...],
                       preferred_element_type=jnp.float32)
             + b_ref[...])
        h_ref[...] = h
        lg_ref[...] = jnp.dot(h, wc_ref[...],
                              preferred_element_type=jnp.float32) + bc_ref[...]

    return pl.pallas_call(
        body,
        grid=(n // bn,),
        in_specs=[
            pl.BlockSpec((2, bn, dout), lambda i: (0, i, 0)),
            pl.BlockSpec((bn, din), lambda i: (i, 0)),
            pl.BlockSpec((din, dout), lambda i: (0, 0)),
            pl.BlockSpec((1, dout), lambda i: (0, 0)),
            pl.BlockSpec((dout, dc), lambda i: (0, 0)),
            pl.BlockSpec((1, dc), lambda i: (0, 0)),
        ],
        out_specs=[
            pl.BlockSpec((bn, dout), lambda i: (i, 0)),
            pl.BlockSpec((bn, dc), lambda i: (i, 0)),
        ],
        out_shape=[
            jax.ShapeDtypeStruct((n, dout), jnp.float32),
            jax.ShapeDtypeStruct((n, dc), jnp.float32),
        ],
    )(p, x, root, b, wc, bc)


def kernel(x, edge_index, edge_type, W1, root1, b1, W2, root2, b2, Wc, bc):
    n, din = x.shape
    r = W1.shape[0]
    d = W1.shape[2]
    e = edge_type.shape[0]
    nc = Wc.shape[1]

    cpt = -(-e // (NW * CH))
    e_pad = NW * cpt * CH
    pad = e_pad - e
    pad_seg = n * r
    cnt_pad = pad_seg + 1
    if cnt_pad % (NSUB * 16):
        cnt_pad += NSUB * 16 - cnt_pad % (NSUB * 16)
    n_pad = NSUB * (-(-n // (NSUB * 8)) * 8)

    src = edge_index[0]
    dst = edge_index[1]
    et = edge_type.astype(jnp.int32)
    gidx = src * r + et
    seg = dst * r + et
    i0 = jnp.zeros((pad,), jnp.int32)
    gidx3 = jnp.concatenate([gidx, i0]).reshape(NW, cpt, CH)
    seg3 = jnp.concatenate([seg, jnp.full((pad,), pad_seg, jnp.int32)]
                           ).reshape(NW, cpt, CH)
    dst3 = jnp.concatenate([dst, i0]).reshape(NW, cpt, CH)

    w3 = _make_weights_kernel(cpt, cnt_pad, pad_seg)(seg3)
    agg = _make_agg_kernel(n_pad, d, cpt)

    bn = 1000
    b1_2d = b1.reshape(1, d)
    b2_2d = b2.reshape(1, d)
    wc_pad = jnp.zeros((d, d), jnp.float32).at[:, :nc].set(Wc)
    bc_pad = jnp.zeros((1, d), jnp.float32).at[:, :nc].set(bc)

    xw1 = _relmm(x, W1, bn).reshape(n * r, d)
    p1 = agg(xw1, gidx3, dst3, w3)
    h, xw2_3d = _post_relmm(p1, x, root1, b1_2d, W2, bn)

    p2 = agg(xw2_3d.reshape(n * r, d), gidx3, dst3, w3)
    h2, logits_pad = _post_final(p2, h, root2, b2_2d, wc_pad, bc_pad, bn)
    return (h2, logits_pad[:, :nc])


# speedup vs baseline: 3.6046x; 1.0060x over previous
import functools
import jax
import jax.numpy as jnp
from jax import lax
from jax.experimental import pallas as pl
from jax.experimental.pallas import tpu as pltpu
from jax.experimental.pallas import tpu_sc as plsc

NCORES = 2
NSUB = 16
NW = NCORES * NSUB
CH = 128


def _sc_mesh():
    return plsc.VectorSubcoreMesh(
        core_axis_name="c", subcore_axis_name="s",
        num_cores=NCORES, num_subcores=NSUB)


def _make_weights_kernel(cpt, cnt_pad, pad_seg):
    zpt = cnt_pad // NSUB

    @functools.partial(
        pl.kernel,
        out_type=jax.ShapeDtypeStruct((NW, cpt, CH), jnp.float32),
        mesh=_sc_mesh(),
        scratch_types=[
            pltpu.VMEM((NCORES, cpt, CH), jnp.int32),
            pltpu.VMEM((CH,), jnp.float32),
            pltpu.VMEM((2, CH), jnp.float32),
            pltpu.VMEM((2, CH), jnp.float32),
            pltpu.VMEM((zpt,), jnp.float32),
            pltpu.SemaphoreType.DMA,
            pltpu.SemaphoreType.DMA((2,)),
            pltpu.SemaphoreType.DMA((2,)),
            pltpu.VMEM_SHARED((cnt_pad,), jnp.float32),
        ],
    )
    def k(seg_hbm, w_hbm, seg_v, ones_v, cval_r, wout_r, zz_v,
          sem_c, sem_cv, sem_wo, cnt_sh):
        cid = lax.axis_index("c")
        sid = lax.axis_index("s")
        row = NCORES * sid + cid
        for l in range(CH // 16):
            ones_v[pl.ds(l * 16, 16)] = jnp.full((16,), 1.0, jnp.float32)

        def zfill(i, _):
            zz_v[pl.ds(i * 16, 16)] = jnp.zeros((16,), jnp.float32)
            return 0
        lax.fori_loop(0, zpt // 16, zfill, 0)
        for t in range(NCORES):
            pltpu.sync_copy(seg_hbm.at[NCORES * sid + t], seg_v.at[t])
        pltpu.sync_copy(zz_v, cnt_sh.at[pl.ds(sid * zpt, zpt)])
        plsc.subcore_barrier()

        def count_body(j, _):
            for t in range(NCORES):
                pltpu.async_copy(ones_v, cnt_sh.at[seg_v.at[t, j]], sem_c,
                                 add=True)
            return 0
        lax.fori_loop(0, cpt, count_body, 0)

        def count_drain(j, _):
            for t in range(NCORES):
                pltpu.make_async_copy(ones_v, cnt_sh.at[seg_v.at[t, j]],
                                      sem_c).wait()
            return 0
        lax.fori_loop(0, cpt, count_drain, 0)
        plsc.subcore_barrier()

        def wpre(j, p):
            pltpu.async_copy(cnt_sh.at[seg_v.at[cid, j]], cval_r.at[p],
                             sem_cv.at[p])

        def wprime(p0, _):
            wpre(p0, p0)
            return 0
        lax.fori_loop(0, 2, wprime, 0)

        def w_body(j, _):
            p = lax.rem(j, 2)
            pltpu.make_async_copy(cnt_sh.at[seg_v.at[cid, j]], cval_r.at[p],
                                  sem_cv.at[p]).wait()

            @pl.when(j >= 2)
            def _():
                pltpu.make_async_copy(wout_r.at[p], w_hbm.at[row, j],
                                      sem_wo.at[p]).wait()
            for l in range(CH // 16):
                sl = pl.ds(l * 16, 16)
                c = cval_r[p, sl]
                s = seg_v[cid, j, sl]
                w = 1.0 / jnp.maximum(c, 1.0)
                wout_r[p, sl] = jnp.where(s >= pad_seg, 0.0, w)
            pltpu.async_copy(wout_r.at[p], w_hbm.at[row, j], sem_wo.at[p])

            @pl.when(j + 2 < cpt)
            def _():
                wpre(j + 2, p)
            return 0
        lax.fori_loop(0, cpt, w_body, 0)

        def wdrain(q, _):
            j = cpt - 2 + q
            p = lax.rem(j, 2)
            pltpu.make_async_copy(wout_r.at[p], w_hbm.at[row, j],
                                  sem_wo.at[p]).wait()
            return 0
        lax.fori_loop(0, 2, wdrain, 0)

    return k


def _make_agg_kernel(n_pad, d, cpt):
    zs = n_pad // NSUB

    @functools.partial(
        pl.kernel,
        out_type=jax.ShapeDtypeStruct((NCORES, n_pad, d), jnp.float32),
        mesh=_sc_mesh(),
        scratch_types=[
            pltpu.VMEM((cpt, CH), jnp.int32),
            pltpu.VMEM((2, CH), jnp.int32),
            pltpu.VMEM((2, CH), jnp.float32),
            pltpu.VMEM((2, CH, d), jnp.float32),
            pltpu.SemaphoreType.DMA((2,)),
            pltpu.SemaphoreType.DMA((2,)),
            pltpu.SemaphoreType.DMA((2,)),
            pltpu.VMEM_SHARED((n_pad, d), jnp.float32),
        ],
        compiler_params=pltpu.CompilerParams(needs_layout_passes=False),
    )
    def k(xw_hbm, gidx_hbm, dst_hbm, w_hbm, out_hbm,
          gidx_v, dst_r, w_r, rows_b, sem_g, sem_d, sem_w, acc_sh):
        cid = lax.axis_index("c")
        sid = lax.axis_index("s")
        wid = NCORES * sid + cid
        pltpu.sync_copy(gidx_hbm.at[wid], gidx_v)

        def zfill(i, _):
            for g in range(d // 16):
                rows_b[0, i, pl.ds(g * 16, 16)] = jnp.zeros((16,), jnp.float32)
            return 0
        lax.fori_loop(0, CH, zfill, 0)
        nfull, rem = divmod(zs, CH)
        for q in range(nfull):
            pltpu.sync_copy(rows_b.at[0],
                            acc_sh.at[pl.ds(sid * zs + q * CH, CH)])
        if rem:
            pltpu.sync_copy(rows_b.at[0, pl.ds(0, rem)],
                            acc_sh.at[pl.ds(sid * zs + nfull * CH, rem)])
        plsc.subcore_barrier()

        def prefetch(j, p):
            pltpu.async_copy(xw_hbm.at[gidx_v.at[j]], rows_b.at[p],
                             sem_g.at[p])
            pltpu.async_copy(dst_hbm.at[wid, j], dst_r.at[p], sem_d.at[p])
            pltpu.async_copy(w_hbm.at[wid, j], w_r.at[p], sem_w.at[p])

        def prime(p0, _):
            prefetch(p0, p0)
            return 0
        lax.fori_loop(0, 2, prime, 0)

        def chunk_body(j, _):
            p = lax.rem(j, 2)
            pltpu.make_async_copy(xw_hbm.at[gidx_v.at[j]], rows_b.at[p],
                                  sem_g.at[p]).wait()
            pltpu.make_async_copy(w_hbm.at[wid, j], w_r.at[p],
                                  sem_w.at[p]).wait()
            pv = jnp.full((16,), p, jnp.int32)

            def row_body(i8, _):
                for u in range(8):
                    i = i8 * 8 + u
                    wspl = plsc.load_gather(
                        w_r, [pv, jnp.full((16,), i, jnp.int32)])
                    for g in range(d // 16):
                        sl = pl.ds(g * 16, 16)
                        rows_b[p, i, sl] = rows_b[p, i, sl] * wspl
                return 0
            lax.fori_loop(0, CH // 8, row_body, 0)

            pltpu.make_async_copy(dst_hbm.at[wid, j], dst_r.at[p],
                                  sem_d.at[p]).wait()
            pltpu.sync_copy(rows_b.at[p], acc_sh.at[dst_r.at[p]], add=True)

            @pl.when(j + 2 < cpt)
            def _():
                prefetch(j + 2, p)
            return 0
        lax.fori_loop(0, cpt, chunk_body, 0)
        plsc.subcore_barrier()
        pltpu.sync_copy(acc_sh.at[pl.ds(sid * zs, zs)],
                        out_hbm.at[cid, pl.ds(sid * zs, zs)])

    return k


def _relmm(x, W, bn):
    n, din = x.shape
    r, _, dout = W.shape
    h = dout // 2

    def body(x_ref, w_ref, out_ref):
        xb = x_ref[...]
        for ri in range(r):
            out_ref[:, ri, :] = jnp.dot(xb, w_ref[ri],
                                        preferred_element_type=jnp.float32)

    return pl.pallas_call(
        body,
        grid=(n // bn,),
        in_specs=[
            pl.BlockSpec((bn, din), lambda i: (i, 0)),
            pl.BlockSpec((r, din, dout), lambda i: (0, 0, 0)),
        ],
        out_specs=pl.BlockSpec((bn, r, dout), lambda i: (i, 0, 0)),
        out_shape=jax.ShapeDtypeStruct((n, r, dout), jnp.float32),
    )(x, W)


def _post_relmm(p, x, root, b, W2, bn):
    n, din = x.shape
    n_pad = p.shape[1]
    dout = root.shape[1]
    r = W2.shape[0]

    def body(p_ref, x_ref, root_ref, b_ref, w2_ref, h_ref, xw_ref):
        h = (p_ref[0] + p_ref[1]
             + jnp.dot(x_ref[...], root_ref[...],
                       preferred_element_type=jnp.float32)
             + b_ref[...])
        h = jnp.maximum(h, 0.0)
        h_ref[...] = h
        for ri in range(r):
            xw_ref[:, ri, :] = jnp.dot(h, w2_ref[ri],
                                       preferred_element_type=jnp.float32)

    return pl.pallas_call(
        body,
        grid=(n // bn,),
        in_specs=[
            pl.BlockSpec((2, bn, dout), lambda i: (0, i, 0)),
            pl.BlockSpec((bn, din), lambda i: (i, 0)),
            pl.BlockSpec((din, dout), lambda i: (0, 0)),
            pl.BlockSpec((1, dout), lambda i: (0, 0)),
            pl.BlockSpec((r, dout, dout), lambda i: (0, 0, 0)),
        ],
        out_specs=[
            pl.BlockSpec((bn, dout), lambda i: (i, 0)),
            pl.BlockSpec((bn, r, dout), lambda i: (i, 0, 0)),
        ],
        out_shape=[
            jax.ShapeDtypeStruct((n, dout), jnp.float32),
            jax.ShapeDtypeStruct((n, r, dout), jnp.float32),
        ],
    )(p, x, root, b, W2)


def _post_final(p, x, root, b, wc, bc, bn):
    n, din = x.shape
    dout = root.shape[1]
    dc = wc.shape[1]

    def body(p_ref, x_ref, root_ref, b_ref, wc_ref, bc_ref,
             h_ref, lg_ref):
        h = (p_ref[0] + p_ref[1]
             + jnp.dot(x_ref[...], root_ref[...],
                       preferred_element_type=jnp.float32)
             + b_ref[...])
        h_ref[...] = h
        lg_ref[...] = jnp.dot(h, wc_ref[...],
                              preferred_element_type=jnp.float32) + bc_ref[...]

    return pl.pallas_call(
        body,
        grid=(n // bn,),
        in_specs=[
            pl.BlockSpec((2, bn, dout), lambda i: (0, i, 0)),
            pl.BlockSpec((bn, din), lambda i: (i, 0)),
            pl.BlockSpec((din, dout), lambda i: (0, 0)),
            pl.BlockSpec((1, dout), lambda i: (0, 0)),
            pl.BlockSpec((dout, dc), lambda i: (0, 0)),
            pl.BlockSpec((1, dc), lambda i: (0, 0)),
        ],
        out_specs=[
            pl.BlockSpec((bn, dout), lambda i: (i, 0)),
            pl.BlockSpec((bn, dc), lambda i: (i, 0)),
        ],
        out_shape=[
            jax.ShapeDtypeStruct((n, dout), jnp.float32),
            jax.ShapeDtypeStruct((n, dc), jnp.float32),
        ],
    )(p, x, root, b, wc, bc)


def kernel(x, edge_index, edge_type, W1, root1, b1, W2, root2, b2, Wc, bc):
    n, din = x.shape
    r = W1.shape[0]
    d = W1.shape[2]
    e = edge_type.shape[0]
    nc = Wc.shape[1]

    cpt = -(-e // (NW * CH))
    e_pad = NW * cpt * CH
    pad = e_pad - e
    pad_seg = n * r
    cnt_pad = pad_seg + 1
    if cnt_pad % (NSUB * 16):
        cnt_pad += NSUB * 16 - cnt_pad % (NSUB * 16)
    n_pad = NSUB * (-(-n // (NSUB * 8)) * 8)

    src = edge_index[0]
    dst = edge_index[1]
    et = edge_type.astype(jnp.int32)
    gidx = src * r + et
    seg = dst * r + et
    i0 = jnp.zeros((pad,), jnp.int32)
    gidx3 = jnp.concatenate([gidx, i0]).reshape(NW, cpt, CH)
    seg3 = jnp.concatenate([seg, jnp.full((pad,), pad_seg, jnp.int32)]
                           ).reshape(NW, cpt, CH)
    dst3 = jnp.concatenate([dst, i0]).reshape(NW, cpt, CH)

    w3 = _make_weights_kernel(cpt, cnt_pad, pad_seg)(seg3)
    agg = _make_agg_kernel(n_pad, d, cpt)

    bn = 2000
    b1_2d = b1.reshape(1, d)
    b2_2d = b2.reshape(1, d)
    wc_pad = jnp.zeros((d, d), jnp.float32).at[:, :nc].set(Wc)
    bc_pad = jnp.zeros((1, d), jnp.float32).at[:, :nc].set(bc)

    xw1 = _relmm(x, W1, bn).reshape(n * r, d)
    p1 = agg(xw1, gidx3, dst3, w3)
    h, xw2_3d = _post_relmm(p1, x, root1, b1_2d, W2, bn)

    p2 = agg(xw2_3d.reshape(n * r, d), gidx3, dst3, w3)
    h2, logits_pad = _post_final(p2, h, root2, b2_2d, wc_pad, bc_pad, bn)
    return (h2, logits_pad[:, :nc])
